# baseline (speedup 1.0000x reference)
import functools

import jax
import jax.numpy as jnp
from jax import lax
from jax.experimental import pallas as pl
from jax.experimental.pallas import tpu as pltpu
from jax.experimental.pallas import tpu_sc as plsc

H = 128
L = 6
N = 10000
E = 320000
ALPHA = 0.1

NPAD = 10240
NCORES = 2
NSUB = 16
NTILES = NCORES * NSUB
CH = 128
NBUF = 2
NPASS = 2
NCHUNK = -(-E // (NTILES * CH * NBUF * NPASS)) * NBUF * NPASS
EPT = NCHUNK * CH
EPAD = EPT * NTILES
HC = NCHUNK // NPASS
NGRP = HC // NBUF
RPT = NPAD // NSUB
BROW = 2048
NBLK = NPAD // BROW

_mesh = plsc.VectorSubcoreMesh(core_axis_name="c", subcore_axis_name="s")


def _agg_impl(with_counts, vh, ch, srcs, dsts, zeros, outs, couts, ones,
              acc, sidx, didx, bufs, gsem, ssem):
    c = lax.axis_index("c")
    s = lax.axis_index("s")
    tile = c * NSUB + s
    for a in range(4):
        x = vh if a < 2 else ch
        pltpu.sync_copy(zeros.at[pl.ds(s * RPT, RPT)], acc.at[pl.ds(s * RPT, RPT)])
        plsc.subcore_barrier()
        for p in range(NPASS):
            pltpu.sync_copy(srcs.at[a, tile, pl.ds(p * HC, HC)], sidx)
            pltpu.sync_copy(dsts.at[a, tile, pl.ds(p * HC, HC)], didx)

            for b in range(NBUF):
                pltpu.async_copy(x.at[sidx.at[b]], bufs.at[b], gsem.at[b])

            def group(g, carry):
                for b in range(NBUF):
                    j = g * NBUF + b
                    pltpu.make_async_copy(x.at[sidx.at[j]], bufs.at[b],
                                          gsem.at[b]).wait()
                    pltpu.async_copy(bufs.at[b], acc.at[didx.at[j]],
                                     ssem.at[b], add=True).wait()
                    pltpu.async_copy(x.at[sidx.at[j + NBUF]], bufs.at[b],
                                     gsem.at[b])

                return carry

            lax.fori_loop(0, NGRP - 1, group, 0)
            for b in range(NBUF):
                j = (NGRP - 1) * NBUF + b
                pltpu.make_async_copy(x.at[sidx.at[j]], bufs.at[b],
                                      gsem.at[b]).wait()
                pltpu.async_copy(bufs.at[b], acc.at[didx.at[j]],
                                 ssem.at[b], add=True).wait()
        plsc.subcore_barrier()
        pltpu.sync_copy(acc.at[pl.ds(s * RPT, RPT)],
                        outs[a].at[c, pl.ds(s * RPT, RPT)])
        plsc.subcore_barrier()

    if not with_counts:
        return
    pltpu.sync_copy(ones, bufs.at[0])
    for a in range(4):
        pltpu.sync_copy(zeros.at[pl.ds(s * RPT, RPT)],
                        acc.at[pl.ds(s * RPT, RPT)])
        plsc.subcore_barrier()
        for p in range(NPASS):
            pltpu.sync_copy(dsts.at[a, tile, pl.ds(p * HC, HC)], didx)

            for b in range(NBUF):
                pltpu.async_copy(bufs.at[0], acc.at[didx.at[b]], ssem.at[b],
                                 add=True)

            def cgroup(g, carry):
                for b in range(NBUF):
                    j = (g + 1) * NBUF + b
                    pltpu.make_async_copy(bufs.at[0], acc.at[didx.at[j - NBUF]],
                                          ssem.at[b]).wait()
                    pltpu.async_copy(bufs.at[0], acc.at[didx.at[j]],
                                     ssem.at[b], add=True)
                return carry

            lax.fori_loop(0, NGRP - 1, cgroup, 0)
            for b in range(NBUF):
                j = (NGRP - 1) * NBUF + b
                pltpu.make_async_copy(bufs.at[0], acc.at[didx.at[j]],
                                      ssem.at[b]).wait()
        plsc.subcore_barrier()
        pltpu.sync_copy(acc.at[pl.ds(s * RPT, RPT)],
                        couts[a].at[c, pl.ds(s * RPT, RPT)])
        plsc.subcore_barrier()


def _agg_first_body(vh, ch, srcs, dsts, zeros, ones,
                    o0, o1, o2, o3, c0, c1, c2, c3,
                    acc, sidx, didx, bufs, gsem, ssem):
    _agg_impl(True, vh, ch, srcs, dsts, zeros, [o0, o1, o2, o3],
              [c0, c1, c2, c3], ones, acc, sidx, didx, bufs, gsem, ssem)


def _agg_rest_body(vh, ch, srcs, dsts, zeros, o0, o1, o2, o3,
                   acc, sidx, didx, bufs, gsem, ssem):
    _agg_impl(False, vh, ch, srcs, dsts, zeros, [o0, o1, o2, o3],
              None, None, acc, sidx, didx, bufs, gsem, ssem)


_agg_scratch = [
    pltpu.VMEM_SHARED((NPAD, H), jnp.float32),
    pltpu.VMEM((HC, CH), jnp.int32),
    pltpu.VMEM((HC, CH), jnp.int32),
    pltpu.VMEM((NBUF, CH, H), jnp.float32),
    pltpu.SemaphoreType.DMA((NBUF,)),
    pltpu.SemaphoreType.DMA((NBUF,)),
]

_agg_first_call = pl.kernel(
    _agg_first_body,
    out_type=[jax.ShapeDtypeStruct((NCORES, NPAD, H), jnp.float32)
              for _ in range(8)],
    mesh=_mesh,
    scratch_types=list(_agg_scratch),
)

_agg_call = pl.kernel(
    _agg_rest_body,
    out_type=[jax.ShapeDtypeStruct((NCORES, NPAD, H), jnp.float32)
              for _ in range(4)],
    mesh=_mesh,
    scratch_types=list(_agg_scratch),
)


def _enc_body(xv, xc, Wv, bv, Wc, bc, vh0_o, ch0_o):
    vh0_o[...] = jax.nn.relu(
        jnp.dot(xv[...], Wv[...], preferred_element_type=jnp.float32,
                precision=lax.Precision.HIGHEST) + bv[...])
    ch0_o[...] = jax.nn.relu(xc[...] * Wc[...] + bc[...])


def _make_enc():
    bspec_row = lambda w: pl.BlockSpec((BROW, w), lambda i: (i, 0))
    bspec_full = lambda a, b: pl.BlockSpec((a, b), lambda i: (0, 0))
    return pl.pallas_call(
        _enc_body,
        grid=(NBLK,),
        in_specs=[bspec_row(4), bspec_row(1), bspec_full(4, H),
                  bspec_full(1, H), bspec_full(1, H), bspec_full(1, H)],
        out_specs=[bspec_row(H), bspec_row(H)],
        out_shape=[jax.ShapeDtypeStruct((NPAD, H), jnp.float32),
                   jax.ShapeDtypeStruct((NPAD, H), jnp.float32)],
    )


def _layer_math(aCp, aCn, aVp, aVn, ipd, ind, ips, ins, vh, ch, vh0, ch0,
                Wl, bl, Wr):
    dot = functools.partial(jnp.dot, preferred_element_type=jnp.float32,
                            precision=lax.Precision.HIGHEST)

    def mean(aref, cref):
        inv = 1.0 / jnp.maximum(cref[0] + cref[1], 1.0)
        return (aref[0] + aref[1]) * inv

    mcp = mean(aCp, ipd)
    mcn = mean(aCn, ind)
    mvp = mean(aVp, ips)
    mvn = mean(aVn, ins)
    out_c = (dot(mcp, Wl[0]) + dot(mcn, Wl[1]) + dot(ch[...], Wr[0] + Wr[1])
             + (bl[0:1] + bl[1:2]))
    out_v = (dot(mvp, Wl[2]) + dot(mvn, Wl[3]) + dot(vh[...], Wr[2] + Wr[3])
             + (bl[2:3] + bl[3:4]))
    c_new = jax.nn.relu((1.0 - ALPHA) * out_c + ALPHA * ch0[...] + ch[...])
    v_new = jax.nn.relu((1.0 - ALPHA) * out_v + ALPHA * vh0[...] + vh[...])
    return v_new, c_new


def _layer_body(aCp, aCn, aVp, aVn, ipd, ind, ips, ins, vh, ch, vh0, ch0,
                Wl, bl, Wr, vo, co):
    v_new, c_new = _layer_math(aCp, aCn, aVp, aVn, ipd, ind, ips, ins,
                               vh, ch, vh0, ch0, Wl, bl, Wr)
    vo[...] = v_new
    co[...] = c_new


def _final_body(aCp, aCn, aVp, aVn, ipd, ind, ips, ins, vh, ch, vh0, ch0,
                Wl, bl, Wr, wfT, bf, yo):
    v_new, _ = _layer_math(aCp, aCn, aVp, aVn, ipd, ind, ips, ins,
                           vh, ch, vh0, ch0, Wl, bl, Wr)
    yo[...] = jnp.sum(v_new * wfT[...], axis=1, keepdims=True) + bf[...]


def _make_layer(final):
    bspec_row = pl.BlockSpec((BROW, H), lambda i: (i, 0))
    bspec_agg = pl.BlockSpec((NCORES, BROW, H), lambda i: (0, i, 0))
    bspec_icnt = pl.BlockSpec((NCORES, BROW, 1), lambda i: (0, i, 0))
    in_specs = ([bspec_agg] * 4 + [bspec_icnt] * 4 + [bspec_row] * 4
                + [pl.BlockSpec((4, H, H), lambda i: (0, 0, 0)),
                   pl.BlockSpec((4, H), lambda i: (0, 0)),
                   pl.BlockSpec((4, H, H), lambda i: (0, 0, 0))])
    if final:
        in_specs += [pl.BlockSpec((1, H), lambda i: (0, 0)),
                     pl.BlockSpec((1, 1), lambda i: (0, 0))]
        return pl.pallas_call(
            _final_body, grid=(NBLK,), in_specs=in_specs,
            out_specs=[pl.BlockSpec((BROW, 1), lambda i: (i, 0))],
            out_shape=[jax.ShapeDtypeStruct((NPAD, 1), jnp.float32)],
        )
    return pl.pallas_call(
        _layer_body, grid=(NBLK,), in_specs=in_specs,
        out_specs=[bspec_row, bspec_row],
        out_shape=[jax.ShapeDtypeStruct((NPAD, H), jnp.float32),
                   jax.ShapeDtypeStruct((NPAD, H), jnp.float32)],
    )


def kernel(x_variable, x_clause, edge_index_pos, edge_index_neg,
           Wv, bv, Wc, bc, Wl, bl, Wr, Wf, bf):
    f32 = jnp.float32
    ps, pd = edge_index_pos[0], edge_index_pos[1]
    ns, nd = edge_index_neg[0], edge_index_neg[1]

    npad = EPAD - E
    pad_src = jnp.arange(npad, dtype=jnp.int32) % N
    pad_dst = N + jnp.arange(npad, dtype=jnp.int32) % (NPAD - N)

    def padi(x, v):
        return jnp.concatenate([x, v])

    srcs = jnp.stack([padi(ps, pad_src), padi(ns, pad_src),
                      padi(pd, pad_src), padi(nd, pad_src)])
    dsts = jnp.stack([padi(pd, pad_dst), padi(nd, pad_dst),
                      padi(ps, pad_dst), padi(ns, pad_dst)])
    srcs = srcs.reshape(4, NTILES, NCHUNK, CH)
    dsts = dsts.reshape(4, NTILES, NCHUNK, CH)

    xv = jnp.pad(x_variable, ((0, NPAD - N), (0, 0)))
    xc = jnp.pad(x_clause, ((0, NPAD - N), (0, 0)))
    zeros = jnp.zeros((NPAD, H), f32)
    ones = jnp.ones((CH, H), f32)

    vh0, ch0 = _make_enc()(xv, xc, Wv, bv.reshape(1, H), Wc,
                           bc.reshape(1, H))

    vh, ch = vh0, ch0
    layer_call = _make_layer(False)
    final_call = _make_layer(True)
    for i in range(L):
        if i == 0:
            (a0, a1, a2, a3, craw0, craw1, craw2, craw3) = _agg_first_call(
                vh, ch, srcs, dsts, zeros, ones)
            cnt_slices = [c[:, :, 0:1] for c in
                          (craw0, craw1, craw2, craw3)]
        else:
            a0, a1, a2, a3 = _agg_call(vh, ch, srcs, dsts, zeros)
        args = (a0, a1, a2, a3, *cnt_slices, vh, ch, vh0, ch0,
                Wl[i], bl[i], Wr[i])
        if i < L - 1:
            vh, ch = layer_call(*args)
        else:
            (y,) = final_call(*args, Wf.reshape(1, H), bf.reshape(1, 1))
    return y[:N]


# speedup vs baseline: 1.0141x; 1.0141x over previous
import functools

import jax
import jax.numpy as jnp
from jax import lax
from jax.experimental import pallas as pl
from jax.experimental.pallas import tpu as pltpu
from jax.experimental.pallas import tpu_sc as plsc

H = 128
L = 6
N = 10000
E = 320000
ALPHA = 0.1

NPAD = 10240
NCORES = 2
NSUB = 16
NTILES = NCORES * NSUB
CH = 128
NBUF = 2
NPASS = 2
NCHUNK = -(-E // (NTILES * CH * NBUF * NPASS)) * NBUF * NPASS
EPT = NCHUNK * CH
EPAD = EPT * NTILES
HC = NCHUNK // NPASS
NGRP = HC // NBUF
RPT = NPAD // NSUB
BROW = 2048
NBLK = NPAD // BROW

_mesh = plsc.VectorSubcoreMesh(core_axis_name="c", subcore_axis_name="s")


def _agg_impl(with_counts, vh, ch, srcs, dsts, zeros, outs, couts, ones,
              acc, sidx, didx, bufs, gsem, ssem):
    c = lax.axis_index("c")
    s = lax.axis_index("s")
    tile = c * NSUB + s
    for a in range(4):
        x = vh if a < 2 else ch
        pltpu.sync_copy(zeros.at[pl.ds(s * RPT, RPT)], acc.at[pl.ds(s * RPT, RPT)])
        plsc.subcore_barrier()
        for p in range(NPASS):
            pltpu.sync_copy(srcs.at[a, tile, pl.ds(p * HC, HC)], sidx)
            pltpu.sync_copy(dsts.at[a, tile, pl.ds(p * HC, HC)], didx)

            for b in range(NBUF):
                pltpu.async_copy(x.at[sidx.at[b]], bufs.at[b], gsem.at[b])

            def group(g, carry):
                for b in range(NBUF):
                    j = g * NBUF + b
                    pltpu.make_async_copy(x.at[sidx.at[j]], bufs.at[b],
                                          gsem.at[b]).wait()
                    pltpu.async_copy(bufs.at[b], acc.at[didx.at[j]],
                                     ssem.at[b], add=True).wait()
                    pltpu.async_copy(x.at[sidx.at[j + NBUF]], bufs.at[b],
                                     gsem.at[b])

                return carry

            lax.fori_loop(0, NGRP - 1, group, 0)
            for b in range(NBUF):
                j = (NGRP - 1) * NBUF + b
                pltpu.make_async_copy(x.at[sidx.at[j]], bufs.at[b],
                                      gsem.at[b]).wait()
                pltpu.async_copy(bufs.at[b], acc.at[didx.at[j]],
                                 ssem.at[b], add=True).wait()
        plsc.subcore_barrier()
        pltpu.sync_copy(acc.at[pl.ds(s * RPT, RPT)],
                        outs[a].at[c, pl.ds(s * RPT, RPT)])
        plsc.subcore_barrier()

    if not with_counts:
        return
    pltpu.sync_copy(ones, bufs.at[0])
    for a in range(4):
        pltpu.sync_copy(zeros.at[pl.ds(s * RPT, RPT)],
                        acc.at[pl.ds(s * RPT, RPT)])
        plsc.subcore_barrier()
        for p in range(NPASS):
            pltpu.sync_copy(dsts.at[a, tile, pl.ds(p * HC, HC)], didx)

            for b in range(NBUF):
                pltpu.async_copy(bufs.at[0], acc.at[didx.at[b]], ssem.at[b],
                                 add=True)

            def cgroup(g, carry):
                for b in range(NBUF):
                    j = (g + 1) * NBUF + b
                    pltpu.make_async_copy(bufs.at[0], acc.at[didx.at[j - NBUF]],
                                          ssem.at[b]).wait()
                    pltpu.async_copy(bufs.at[0], acc.at[didx.at[j]],
                                     ssem.at[b], add=True)
                return carry

            lax.fori_loop(0, NGRP - 1, cgroup, 0)
            for b in range(NBUF):
                j = (NGRP - 1) * NBUF + b
                pltpu.make_async_copy(bufs.at[0], acc.at[didx.at[j]],
                                      ssem.at[b]).wait()
        plsc.subcore_barrier()
        pltpu.sync_copy(acc.at[pl.ds(s * RPT, RPT)],
                        couts[a].at[c, pl.ds(s * RPT, RPT)])
        plsc.subcore_barrier()


def _agg_first_body(vh, ch, srcs, dsts, zeros, ones,
                    o0, o1, o2, o3, c0, c1, c2, c3,
                    acc, sidx, didx, bufs, gsem, ssem):
    _agg_impl(True, vh, ch, srcs, dsts, zeros, [o0, o1, o2, o3],
              [c0, c1, c2, c3], ones, acc, sidx, didx, bufs, gsem, ssem)


def _agg_rest_body(vh, ch, srcs, dsts, zeros, o0, o1, o2, o3,
                   acc, sidx, didx, bufs, gsem, ssem):
    _agg_impl(False, vh, ch, srcs, dsts, zeros, [o0, o1, o2, o3],
              None, None, acc, sidx, didx, bufs, gsem, ssem)


_agg_scratch = [
    pltpu.VMEM_SHARED((NPAD, H), jnp.float32),
    pltpu.VMEM((HC, CH), jnp.int32),
    pltpu.VMEM((HC, CH), jnp.int32),
    pltpu.VMEM((NBUF, CH, H), jnp.float32),
    pltpu.SemaphoreType.DMA((NBUF,)),
    pltpu.SemaphoreType.DMA((NBUF,)),
]

_agg_first_call = pl.kernel(
    _agg_first_body,
    out_type=[jax.ShapeDtypeStruct((NCORES, NPAD, H), jnp.float32)
              for _ in range(8)],
    mesh=_mesh,
    scratch_types=list(_agg_scratch),
)

_agg_call = pl.kernel(
    _agg_rest_body,
    out_type=[jax.ShapeDtypeStruct((NCORES, NPAD, H), jnp.float32)
              for _ in range(4)],
    mesh=_mesh,
    scratch_types=list(_agg_scratch),
)


def _enc_body(xv, xc, Wv, bv, Wc, bc, vh0_o, ch0_o):
    vh0_o[...] = jax.nn.relu(
        jnp.dot(xv[...], Wv[...], preferred_element_type=jnp.float32)
        + bv[...])
    ch0_o[...] = jax.nn.relu(xc[...] * Wc[...] + bc[...])


def _make_enc():
    bspec_row = lambda w: pl.BlockSpec((BROW, w), lambda i: (i, 0))
    bspec_full = lambda a, b: pl.BlockSpec((a, b), lambda i: (0, 0))
    return pl.pallas_call(
        _enc_body,
        grid=(NBLK,),
        in_specs=[bspec_row(4), bspec_row(1), bspec_full(4, H),
                  bspec_full(1, H), bspec_full(1, H), bspec_full(1, H)],
        out_specs=[bspec_row(H), bspec_row(H)],
        out_shape=[jax.ShapeDtypeStruct((NPAD, H), jnp.float32),
                   jax.ShapeDtypeStruct((NPAD, H), jnp.float32)],
    )


def _layer_math(aCp, aCn, aVp, aVn, ipd, ind, ips, ins, vh, ch, vh0, ch0,
                Wl, bl, Wr):
    dot = functools.partial(jnp.dot, preferred_element_type=jnp.float32)

    def mean(aref, cref):
        inv = 1.0 / jnp.maximum(cref[0] + cref[1], 1.0)
        return (aref[0] + aref[1]) * inv

    mcp = mean(aCp, ipd)
    mcn = mean(aCn, ind)
    mvp = mean(aVp, ips)
    mvn = mean(aVn, ins)
    out_c = (dot(mcp, Wl[0]) + dot(mcn, Wl[1]) + dot(ch[...], Wr[0] + Wr[1])
             + (bl[0:1] + bl[1:2]))
    out_v = (dot(mvp, Wl[2]) + dot(mvn, Wl[3]) + dot(vh[...], Wr[2] + Wr[3])
             + (bl[2:3] + bl[3:4]))
    c_new = jax.nn.relu((1.0 - ALPHA) * out_c + ALPHA * ch0[...] + ch[...])
    v_new = jax.nn.relu((1.0 - ALPHA) * out_v + ALPHA * vh0[...] + vh[...])
    return v_new, c_new


def _layer_body(aCp, aCn, aVp, aVn, ipd, ind, ips, ins, vh, ch, vh0, ch0,
                Wl, bl, Wr, vo, co):
    v_new, c_new = _layer_math(aCp, aCn, aVp, aVn, ipd, ind, ips, ins,
                               vh, ch, vh0, ch0, Wl, bl, Wr)
    vo[...] = v_new
    co[...] = c_new


def _final_body(aCp, aCn, aVp, aVn, ipd, ind, ips, ins, vh, ch, vh0, ch0,
                Wl, bl, Wr, wfT, bf, yo):
    v_new, _ = _layer_math(aCp, aCn, aVp, aVn, ipd, ind, ips, ins,
                           vh, ch, vh0, ch0, Wl, bl, Wr)
    yo[...] = jnp.sum(v_new * wfT[...], axis=1, keepdims=True) + bf[...]


def _make_layer(final):
    bspec_row = pl.BlockSpec((BROW, H), lambda i: (i, 0))
    bspec_agg = pl.BlockSpec((NCORES, BROW, H), lambda i: (0, i, 0))
    bspec_icnt = pl.BlockSpec((NCORES, BROW, 1), lambda i: (0, i, 0))
    in_specs = ([bspec_agg] * 4 + [bspec_icnt] * 4 + [bspec_row] * 4
                + [pl.BlockSpec((4, H, H), lambda i: (0, 0, 0)),
                   pl.BlockSpec((4, H), lambda i: (0, 0)),
                   pl.BlockSpec((4, H, H), lambda i: (0, 0, 0))])
    if final:
        in_specs += [pl.BlockSpec((1, H), lambda i: (0, 0)),
                     pl.BlockSpec((1, 1), lambda i: (0, 0))]
        return pl.pallas_call(
            _final_body, grid=(NBLK,), in_specs=in_specs,
            out_specs=[pl.BlockSpec((BROW, 1), lambda i: (i, 0))],
            out_shape=[jax.ShapeDtypeStruct((NPAD, 1), jnp.float32)],
        )
    return pl.pallas_call(
        _layer_body, grid=(NBLK,), in_specs=in_specs,
        out_specs=[bspec_row, bspec_row],
        out_shape=[jax.ShapeDtypeStruct((NPAD, H), jnp.float32),
                   jax.ShapeDtypeStruct((NPAD, H), jnp.float32)],
    )


def kernel(x_variable, x_clause, edge_index_pos, edge_index_neg,
           Wv, bv, Wc, bc, Wl, bl, Wr, Wf, bf):
    f32 = jnp.float32
    ps, pd = edge_index_pos[0], edge_index_pos[1]
    ns, nd = edge_index_neg[0], edge_index_neg[1]

    npad = EPAD - E
    pad_src = jnp.arange(npad, dtype=jnp.int32) % N
    pad_dst = N + jnp.arange(npad, dtype=jnp.int32) % (NPAD - N)

    def padi(x, v):
        return jnp.concatenate([x, v])

    srcs = jnp.stack([padi(ps, pad_src), padi(ns, pad_src),
                      padi(pd, pad_src), padi(nd, pad_src)])
    dsts = jnp.stack([padi(pd, pad_dst), padi(nd, pad_dst),
                      padi(ps, pad_dst), padi(ns, pad_dst)])
    srcs = srcs.reshape(4, NTILES, NCHUNK, CH)
    dsts = dsts.reshape(4, NTILES, NCHUNK, CH)

    xv = jnp.pad(x_variable, ((0, NPAD - N), (0, 0)))
    xc = jnp.pad(x_clause, ((0, NPAD - N), (0, 0)))
    zeros = jnp.zeros((NPAD, H), f32)
    ones = jnp.ones((CH, H), f32)

    vh0, ch0 = _make_enc()(xv, xc, Wv, bv.reshape(1, H), Wc,
                           bc.reshape(1, H))

    vh, ch = vh0, ch0
    layer_call = _make_layer(False)
    final_call = _make_layer(True)
    for i in range(L):
        if i == 0:
            (a0, a1, a2, a3, craw0, craw1, craw2, craw3) = _agg_first_call(
                vh, ch, srcs, dsts, zeros, ones)
            cnt_slices = [c[:, :, 0:1] for c in
                          (craw0, craw1, craw2, craw3)]
        else:
            a0, a1, a2, a3 = _agg_call(vh, ch, srcs, dsts, zeros)
        args = (a0, a1, a2, a3, *cnt_slices, vh, ch, vh0, ch0,
                Wl[i], bl[i], Wr[i])
        if i < L - 1:
            vh, ch = layer_call(*args)
        else:
            (y,) = final_call(*args, Wf.reshape(1, H), bf.reshape(1, 1))
    return y[:N]
